# Initial kernel scaffold; baseline (speedup 1.0000x reference)
import jax
import jax.numpy as jnp
from jax.experimental import pallas as pl


def kernel(x, weight):
    raise NotImplementedError("write your pallas kernel here")


# speedup vs baseline: 3.9021x; 3.9021x over previous
import jax
import jax.numpy as jnp
from jax.experimental import pallas as pl
from jax.experimental.pallas import tpu as pltpu

_TOPK = 2
_N_GROUPS = 2
_ROUTE_SCALE = 2.5
_BT = 1024


def _router_block(x_ref, w_ref, wout_ref, iout_ref):
    bt = x_ref.shape[0]
    ne = w_ref.shape[0]
    gsize = ne // _N_GROUPS
    logits = jax.lax.dot_general(
        x_ref[...], w_ref[...], (((1,), (1,)), ((), ())),
        preferred_element_type=jnp.float32,
    )
    s = 1.0 / (1.0 + jnp.exp(-logits))
    col = jax.lax.broadcasted_iota(jnp.int32, (bt, ne), 1)
    grp = col // gsize
    neg = jnp.float32(-jnp.inf)
    g0 = jnp.max(jnp.where(grp == 0, s, neg), axis=1, keepdims=True)
    g1 = jnp.max(jnp.where(grp == 1, s, neg), axis=1, keepdims=True)
    chosen = jnp.where(g0 >= g1, 0, 1)
    m = jnp.where(grp == chosen, s, neg)
    v1 = jnp.max(m, axis=1, keepdims=True)
    i1 = jnp.min(jnp.where(m == v1, col, ne), axis=1, keepdims=True)
    m2 = jnp.where(col == i1, neg, m)
    v2 = jnp.max(m2, axis=1, keepdims=True)
    i2 = jnp.min(jnp.where(m2 == v2, col, ne), axis=1, keepdims=True)
    inv = _ROUTE_SCALE / (v1 + v2)
    wout_ref[:, 0:1] = v1 * inv
    wout_ref[:, 1:2] = v2 * inv
    iout_ref[:, 0:1] = i1
    iout_ref[:, 1:2] = i2


@jax.jit
def kernel(x, weight):
    t, dim = x.shape
    ne = weight.shape[0]
    grid = (t // _BT,)
    wout, iout = pl.pallas_call(
        _router_block,
        grid=grid,
        in_specs=[
            pl.BlockSpec((_BT, dim), lambda i: (i, 0)),
            pl.BlockSpec((ne, dim), lambda i: (0, 0)),
        ],
        out_specs=[
            pl.BlockSpec((_BT, _TOPK), lambda i: (i, 0)),
            pl.BlockSpec((_BT, _TOPK), lambda i: (i, 0)),
        ],
        out_shape=[
            jax.ShapeDtypeStruct((t, _TOPK), jnp.float32),
            jax.ShapeDtypeStruct((t, _TOPK), jnp.int32),
        ],
        compiler_params=pltpu.CompilerParams(
            dimension_semantics=("arbitrary",),
        ),
    )(x, weight)
    return wout.astype(x.dtype), iout


# speedup vs baseline: 4.0610x; 1.0407x over previous
import jax
import jax.numpy as jnp
from jax.experimental import pallas as pl
from jax.experimental.pallas import tpu as pltpu

_TOPK = 2
_N_GROUPS = 2
_ROUTE_SCALE = 2.5
_BT = 2048


def _router_block(x_ref, w_ref, wout_ref, iout_ref):
    bt = x_ref.shape[0]
    ne = w_ref.shape[0]
    gsize = ne // _N_GROUPS
    logits = jax.lax.dot_general(
        x_ref[...], w_ref[...], (((1,), (1,)), ((), ())),
        preferred_element_type=jnp.float32,
    )
    s = 1.0 / (1.0 + jnp.exp(-logits))
    col = jax.lax.broadcasted_iota(jnp.int32, (bt, ne), 1)
    grp = col // gsize
    neg = jnp.float32(-jnp.inf)
    g0 = jnp.max(jnp.where(grp == 0, s, neg), axis=1, keepdims=True)
    g1 = jnp.max(jnp.where(grp == 1, s, neg), axis=1, keepdims=True)
    chosen = jnp.where(g0 >= g1, 0, 1)
    m = jnp.where(grp == chosen, s, neg)
    v1 = jnp.max(m, axis=1, keepdims=True)
    i1 = jnp.min(jnp.where(m == v1, col, ne), axis=1, keepdims=True)
    m2 = jnp.where(col == i1, neg, m)
    v2 = jnp.max(m2, axis=1, keepdims=True)
    i2 = jnp.min(jnp.where(m2 == v2, col, ne), axis=1, keepdims=True)
    inv = _ROUTE_SCALE / (v1 + v2)
    wout_ref[:, 0:1] = v1 * inv
    wout_ref[:, 1:2] = v2 * inv
    iout_ref[:, 0:1] = i1
    iout_ref[:, 1:2] = i2


@jax.jit
def kernel(x, weight):
    t, dim = x.shape
    ne = weight.shape[0]
    grid = (t // _BT,)
    wout, iout = pl.pallas_call(
        _router_block,
        grid=grid,
        in_specs=[
            pl.BlockSpec((_BT, dim), lambda i: (i, 0)),
            pl.BlockSpec((ne, dim), lambda i: (0, 0)),
        ],
        out_specs=[
            pl.BlockSpec((_BT, _TOPK), lambda i: (i, 0)),
            pl.BlockSpec((_BT, _TOPK), lambda i: (i, 0)),
        ],
        out_shape=[
            jax.ShapeDtypeStruct((t, _TOPK), jnp.float32),
            jax.ShapeDtypeStruct((t, _TOPK), jnp.int32),
        ],
        compiler_params=pltpu.CompilerParams(
            dimension_semantics=("arbitrary",),
        ),
    )(x, weight)
    return wout.astype(x.dtype), iout


# speedup vs baseline: 4.2043x; 1.0353x over previous
import jax
import jax.numpy as jnp
from jax.experimental import pallas as pl
from jax.experimental.pallas import tpu as pltpu

_TOPK = 2
_N_GROUPS = 2
_ROUTE_SCALE = 2.5
_BT = 2048


def _router_block(x_ref, w_ref, wout_ref, iout_ref):
    bt = x_ref.shape[0]
    ne = w_ref.shape[0]
    gsize = ne // _N_GROUPS
    logits = jax.lax.dot_general(
        x_ref[...], w_ref[...], (((1,), (1,)), ((), ())),
        preferred_element_type=jnp.float32,
    )
    st = 1.0 / (1.0 + jnp.exp(-logits.T))
    row = jax.lax.broadcasted_iota(jnp.int32, (ne, bt), 0)
    grp = row // gsize
    neg = jnp.float32(-jnp.inf)
    g0 = jnp.max(jnp.where(grp == 0, st, neg), axis=0, keepdims=True)
    g1 = jnp.max(jnp.where(grp == 1, st, neg), axis=0, keepdims=True)
    chosen = jnp.where(g0 >= g1, 0, 1)
    m = jnp.where(grp == chosen, st, neg)
    v1 = jnp.max(m, axis=0, keepdims=True)
    i1 = jnp.min(jnp.where(m == v1, row, ne), axis=0, keepdims=True)
    m2 = jnp.where(row == i1, neg, m)
    v2 = jnp.max(m2, axis=0, keepdims=True)
    i2 = jnp.min(jnp.where(m2 == v2, row, ne), axis=0, keepdims=True)
    inv = _ROUTE_SCALE / (v1 + v2)
    wout_ref[...] = jnp.concatenate([v1 * inv, v2 * inv], axis=0).T
    iout_ref[...] = jnp.concatenate([i1, i2], axis=0).T


@jax.jit
def kernel(x, weight):
    t, dim = x.shape
    ne = weight.shape[0]
    grid = (t // _BT,)
    wout, iout = pl.pallas_call(
        _router_block,
        grid=grid,
        in_specs=[
            pl.BlockSpec((_BT, dim), lambda i: (i, 0)),
            pl.BlockSpec((ne, dim), lambda i: (0, 0)),
        ],
        out_specs=[
            pl.BlockSpec((_BT, _TOPK), lambda i: (i, 0)),
            pl.BlockSpec((_BT, _TOPK), lambda i: (i, 0)),
        ],
        out_shape=[
            jax.ShapeDtypeStruct((t, _TOPK), jnp.float32),
            jax.ShapeDtypeStruct((t, _TOPK), jnp.int32),
        ],
        compiler_params=pltpu.CompilerParams(
            dimension_semantics=("arbitrary",),
        ),
    )(x, weight)
    return wout.astype(x.dtype), iout
